# baseline (speedup 1.0000x reference)
import jax
import jax.numpy as jnp
from jax.experimental import pallas as pl
from jax.experimental.pallas import tpu as pltpu

SMOOTH = 1.0
ALPHA = 0.6
GAMMA = 0.75

_NB = 2


def _loss_kernel(mvp_ref, mvg_ref, cp_ref, cg_ref, map_ref, sums_ref):
    b = pl.program_id(0)

    @pl.when(b == 0)
    def _init():
        sums_ref[0] = 0.0
        sums_ref[1] = 0.0
        sums_ref[2] = 0.0
        sums_ref[3] = 0.0

    vsum = 0.0
    tp = 0.0
    sp = 0.0
    sg = 0.0
    for i in range(_NB):
        d0 = mvg_ref[i, 0] - mvp_ref[i, 0]
        d1 = mvg_ref[i, 1] - mvp_ref[i, 1]
        vmap = d0 * d0 + d1 * d1
        map_ref[i] = vmap
        cp = cp_ref[i, 0]
        cg = cg_ref[i, 0]
        vsum += jnp.sum(vmap)
        tp += jnp.sum(cg * cp)
        sp += jnp.sum(cp)
        sg += jnp.sum(cg)

    sums_ref[0] += vsum
    sums_ref[1] += tp
    sums_ref[2] += sp
    sums_ref[3] += sg


def kernel(hm_pred, match_vectors_pred, conf_masks_pred, hm_gt,
           match_vectors_gt, conf_masks_gt):
    B, C, H, W = match_vectors_pred.shape
    n = B * H * W

    vmap_out, sums = pl.pallas_call(
        _loss_kernel,
        grid=(B // _NB,),
        in_specs=[
            pl.BlockSpec((_NB, C, H, W), lambda b: (b, 0, 0, 0)),
            pl.BlockSpec((_NB, C, H, W), lambda b: (b, 0, 0, 0)),
            pl.BlockSpec((_NB, 1, H, W), lambda b: (b, 0, 0, 0)),
            pl.BlockSpec((_NB, 1, H, W), lambda b: (b, 0, 0, 0)),
        ],
        out_specs=[
            pl.BlockSpec((_NB, H, W), lambda b: (b, 0, 0)),
            pl.BlockSpec(memory_space=pltpu.SMEM),
        ],
        out_shape=[
            jax.ShapeDtypeStruct((B, H, W), jnp.float32),
            jax.ShapeDtypeStruct((4,), jnp.float32),
        ],
    )(match_vectors_pred, match_vectors_gt, conf_masks_pred, conf_masks_gt)

    vec_sum, tp, sum_pred, sum_gt = sums[0], sums[1], sums[2], sums[3]
    fp = sum_pred - tp
    fn = sum_gt - tp
    vector_loss = vec_sum / jnp.float32(n)
    l = (tp + SMOOTH) / jnp.maximum(tp + ALPHA * fn + ((1.0 - ALPHA) * fp + SMOOTH), 1.0)
    conf_loss = jnp.power(1.0 - l, GAMMA)
    loss = 0.9 * vector_loss + 0.1 * conf_loss
    return (loss, vector_loss, conf_loss, vmap_out, tp, fp, fn)
